# baseline (speedup 1.0000x reference)
import functools

import jax
import jax.numpy as jnp
from jax import lax
from jax.experimental import pallas as pl
from jax.experimental.pallas import tpu as pltpu
from jax.experimental.pallas import tpu_sc as plsc

NUM_NODES = 10000
NUM_EDGES = 320000
D_FEAT = 128

NC = 2
NS = 16
NW = NC * NS

EDGES_PER_W = NUM_EDGES // NW
CHUNK = 80
NCHUNKS = EDGES_PER_W // CHUNK
NBUF = 5
NLOOP = NCHUNKS // NBUF


def _make_kernel():
    mesh = plsc.VectorSubcoreMesh(core_axis_name="c", subcore_axis_name="s")

    @functools.partial(
        pl.kernel,
        mesh=mesh,
        out_type=jax.ShapeDtypeStruct((NUM_EDGES, D_FEAT), jnp.float32),
        compiler_params=pltpu.CompilerParams(use_tc_tiling_on_sc=False),
        scratch_types=(
            [pltpu.VMEM((EDGES_PER_W,), jnp.int32)] * 2
            + [pltpu.VMEM((CHUNK, D_FEAT // 2), jnp.int32)] * NBUF
            + [pltpu.VMEM((CHUNK, D_FEAT // 2), jnp.int32)] * NBUF
            + [pltpu.VMEM((CHUNK, D_FEAT), jnp.float32)] * NBUF
            + [pltpu.SemaphoreType.DMA] * (3 * NBUF)
        ),
    )
    def node_to_edge(src_hbm, tgt_hbm, eid_src_hbm, eid_tgt_hbm, out_hbm,
                     *scratch):
        ids_s, ids_t = scratch[0:2]
        rows_s = scratch[2:2 + NBUF]
        rows_t = scratch[2 + NBUF:2 + 2 * NBUF]
        prod = scratch[2 + 2 * NBUF:2 + 3 * NBUF]
        gsem_s = scratch[2 + 3 * NBUF:2 + 4 * NBUF]
        gsem_t = scratch[2 + 4 * NBUF:2 + 5 * NBUF]
        ssem = scratch[2 + 5 * NBUF:2 + 6 * NBUF]

        wid = lax.axis_index("s") * NC + lax.axis_index("c")
        wbase = wid * EDGES_PER_W

        pltpu.sync_copy(eid_src_hbm.at[pl.ds(wbase, EDGES_PER_W)], ids_s)
        pltpu.sync_copy(eid_tgt_hbm.at[pl.ds(wbase, EDGES_PER_W)], ids_t)

        def start_gather(b, c):
            idx_s = ids_s.at[pl.ds(c * CHUNK, CHUNK)]
            idx_t = ids_t.at[pl.ds(c * CHUNK, CHUNK)]
            pltpu.async_copy(src_hbm.at[idx_s], rows_s[b], gsem_s[b])
            pltpu.async_copy(tgt_hbm.at[idx_t], rows_t[b], gsem_t[b])

        def wait_gather(b, c):
            idx_s = ids_s.at[pl.ds(c * CHUNK, CHUNK)]
            idx_t = ids_t.at[pl.ds(c * CHUNK, CHUNK)]
            pltpu.make_async_copy(src_hbm.at[idx_s], rows_s[b], gsem_s[b]).wait()
            pltpu.make_async_copy(tgt_hbm.at[idx_t], rows_t[b], gsem_t[b]).wait()

        def start_store(b, c):
            dst = out_hbm.at[pl.ds(wbase + c * CHUNK, CHUNK)]
            pltpu.async_copy(prod[b], dst, ssem[b])

        def wait_store(b, c):
            dst = out_hbm.at[pl.ds(wbase + c * CHUNK, CHUNK)]
            pltpu.make_async_copy(prod[b], dst, ssem[b]).wait()

        def mul_chunk(b):

            @plsc.parallel_loop(0, CHUNK, unroll=4)
            def mul_body(e):
                for g in range(D_FEAT // 32):
                    wa = rows_s[b][e, pl.ds(g * 16, 16)]
                    wb = rows_t[b][e, pl.ds(g * 16, 16)]
                    a_lo = lax.bitcast_convert_type(wa << 16, jnp.float32)
                    b_lo = lax.bitcast_convert_type(wb << 16, jnp.float32)
                    a_hi = lax.bitcast_convert_type(
                        wa & jnp.int32(-65536), jnp.float32)
                    b_hi = lax.bitcast_convert_type(
                        wb & jnp.int32(-65536), jnp.float32)
                    prod[b][e, pl.ds(g * 32, 16)] = a_lo * b_lo
                    prod[b][e, pl.ds(g * 32 + 16, 16)] = a_hi * b_hi

        for b in range(NBUF):
            start_gather(b, b)

        def loop_body(i, carry):
            for b in range(NBUF):
                c = i * NBUF + b
                pl.when(i >= 1)(lambda: wait_store(b, c - NBUF))
                wait_gather(b, c)
                mul_chunk(b)
                pl.when(i < NLOOP - 1)(lambda: start_gather(b, c + NBUF))
                start_store(b, c)
            return carry

        lax.fori_loop(0, NLOOP, loop_body, 0)

        for b in range(NBUF):
            wait_store(b, NCHUNKS - NBUF + b)

    return node_to_edge


_kernel_fn = _make_kernel()


def kernel(node_src_feats, node_tgt_feats, edge_ids):
    def prep(x):
        n = x.shape[0]
        x = x.reshape(n, D_FEAT // 32, 2, 16)
        x = jnp.swapaxes(x, 2, 3).reshape(n, D_FEAT)
        x = x.astype(jnp.bfloat16)
        return lax.bitcast_convert_type(
            x.reshape(n, D_FEAT // 2, 2), jnp.int32)

    eid_src = edge_ids[0]
    eid_tgt = edge_ids[1]
    return _kernel_fn(prep(node_src_feats), prep(node_tgt_feats),
                      eid_src, eid_tgt)
